# baseline (speedup 1.0000x reference)
import jax
import jax.numpy as jnp
from jax import lax
from jax.experimental import pallas as pl
from jax.experimental.pallas import tpu as pltpu
from jax.experimental.pallas import tpu_sc as plsc

K = 32
C = 384
NS = C // 16
NEG_INF = jnp.float32(-3.4e38)
BIG_I32 = jnp.int32(2**30)


def _sc_topk_body(center, off, x_hbm, idx_hbm, strip_v, idxs_v, sem):
    B = x_hbm.shape[0]
    wid = lax.axis_index("s") * 2 + lax.axis_index("c")

    @pl.when(wid < B)
    def _():
        pltpu.sync_copy(x_hbm.at[wid, :, pl.ds(center - off, 128)], strip_v)
        lane = lax.iota(jnp.int32, 16)
        zero16 = jnp.zeros((16,), jnp.int32)
        ninf_v = jnp.full((16,), NEG_INF, jnp.float32)
        big_v = jnp.full((16,), BIG_I32, jnp.int32)

        slices = []
        for g in range(NS):
            acc = ninf_v
            for i in range(16):
                row = strip_v[g * 16 + i, pl.ds(off, 16)]
                acc = jnp.where(lane == i, jnp.take(row, zero16), acc)
            slices.append(acc)
        slices = tuple(slices)

        def round_body(t, carry):
            prev = carry[0]
            acc = carry[1]
            sl = list(carry[2:])
            bv, bg = ninf_v, big_v
            for j in range(NS):
                gid = lane + j * 16
                v = jnp.where(gid == prev, ninf_v, sl[j])
                sl[j] = v
                take = v > bv
                bv = jnp.where(take, v, bv)
                bg = jnp.where(take, gid, bg)
            for s in (8, 4, 2, 1):
                perm = lax.bitwise_xor(lane, jnp.int32(s))
                pv = jnp.take(bv, perm)
                pg = jnp.take(bg, perm)
                tk = (pv > bv) | ((pv == bv) & (pg < bg))
                bv = jnp.where(tk, pv, bv)
                bg = jnp.where(tk, pg, bg)
            acc = jnp.where(lane == (t & 15), bg, acc)

            @pl.when(t == 15)
            def _():
                idxs_v[pl.ds(0, 16)] = acc

            @pl.when(t == K - 1)
            def _():
                idxs_v[pl.ds(16, 16)] = acc

            return tuple([bg, acc] + sl)

        prev = jnp.full((16,), -1, jnp.int32)
        acc0 = jnp.zeros((16,), jnp.int32)
        lax.fori_loop(0, K, round_body, tuple([prev, acc0]) + slices)
        pltpu.sync_copy(idxs_v, idx_hbm.at[wid])


def _topk_indices(xf, center):
    B = xf.shape[0]
    aligned = (center // 128) * 128
    off = center - aligned
    mesh = plsc.VectorSubcoreMesh(core_axis_name="c", subcore_axis_name="s")

    def body(x_hbm, idx_hbm, strip_v, idxs_v, sem):
        _sc_topk_body(center, off, x_hbm, idx_hbm, strip_v, idxs_v, sem)

    return pl.kernel(
        body,
        out_type=jax.ShapeDtypeStruct((B, K), jnp.int32),
        mesh=mesh,
        scratch_types=[
            pltpu.VMEM((C, 128), jnp.float32),
            pltpu.VMEM((K,), jnp.int32),
            pltpu.SemaphoreType.DMA,
        ],
    )(xf)


def _attend_body(idx_ref, w_ref, bias_ref, x_ref, o_ref):
    b = pl.program_id(0)
    S = o_ref.shape[2]
    acc = jnp.zeros((1, S), jnp.float32)
    for k in range(K):
        c = idx_ref[b, k]
        acc = acc + w_ref[k] * x_ref[:, c, :]
    att = jax.nn.sigmoid(acc + bias_ref[0])
    o_ref[...] = x_ref[...] * att[None]


def kernel(x, conv_w, conv_b):
    B, C_, H, W = x.shape
    S_TOT = H * W
    S = 7168
    n_s = S_TOT // S
    center = (H // 2) * W + (W // 2)

    w = conv_w[0, :, 0, 0]
    xf = x.reshape(B, C_, S_TOT)

    idx = _topk_indices(xf, center)

    out = pl.pallas_call(
        _attend_body,
        grid=(B, n_s),
        out_shape=jax.ShapeDtypeStruct((B, C_, S_TOT), jnp.float32),
        in_specs=[
            pl.BlockSpec(memory_space=pltpu.SMEM),
            pl.BlockSpec(memory_space=pltpu.SMEM),
            pl.BlockSpec(memory_space=pltpu.SMEM),
            pl.BlockSpec((1, C_, S), lambda b, s: (b, 0, s)),
        ],
        out_specs=pl.BlockSpec((1, C_, S), lambda b, s: (b, 0, s)),
        compiler_params=pltpu.CompilerParams(
            dimension_semantics=("parallel", "parallel")),
    )(idx, w, conv_b, xf)
    return out.reshape(B, C_, H, W)
